# baseline (speedup 1.0000x reference)
import jax
import jax.numpy as jnp
from jax import lax
from jax.experimental import pallas as pl
from jax.experimental.pallas import tpu as pltpu
from jax.experimental.pallas import tpu_sc as plsc

B = 1024
S = 200
H = 128
N = B * S
LANES = 16
NUM_WORKERS = 32
PER_W = N // NUM_WORKERS
CHUNK = 128
NCHUNK = PER_W // CHUNK
NSEG = 3
SP = NSEG * S
NBUF = 6


def _segpos_tc_body(seg_ref, pos_ref, out_ref):
    for g in range(NSEG):
        out_ref[g * S:(g + 1) * S, :] = pos_ref[...] + seg_ref[g:g + 1, :]


def _build_segpos(segment_table, pos_table):
    return pl.pallas_call(
        _segpos_tc_body,
        out_shape=jax.ShapeDtypeStruct((SP, H), jnp.float32),
    )(segment_table, pos_table[:S])


def _sc_body(ids_hbm, sids_hbm, tok_hbm, segpos_hbm, out_hbm,
             idx_all, spidx_all, segpos_sh,
             rows_0, rows_1, rows_2, rows_3, rows_4, rows_5,
             tsem_0, tsem_1, tsem_2, tsem_3, tsem_4, tsem_5,
             asem_0, asem_1, asem_2, asem_3, asem_4, asem_5,
             osem_0, osem_1, osem_2, osem_3, osem_4, osem_5):
    info = plsc.get_sparse_core_info()
    nc = info.num_cores
    wid = lax.axis_index("s") * nc + lax.axis_index("c")
    wbase = wid * PER_W
    iota = lax.iota(jnp.int32, LANES)

    rows = (rows_0, rows_1, rows_2, rows_3, rows_4, rows_5)
    tsem = (tsem_0, tsem_1, tsem_2, tsem_3, tsem_4, tsem_5)
    asem = (asem_0, asem_1, asem_2, asem_3, asem_4, asem_5)
    osem = (osem_0, osem_1, osem_2, osem_3, osem_4, osem_5)

    pltpu.sync_copy(ids_hbm.at[pl.ds(wbase, PER_W)], idx_all)
    pltpu.sync_copy(sids_hbm.at[pl.ds(wbase, PER_W)], spidx_all)
    pltpu.sync_copy(segpos_hbm, segpos_sh)

    def spidx_body(i, _):
        sv = spidx_all[pl.ds(i * LANES, LANES)]
        pv = lax.rem(wbase + i * LANES + iota, S)
        spidx_all[pl.ds(i * LANES, LANES)] = sv * S + pv
        return 0

    lax.fori_loop(0, PER_W // LANES, spidx_body, 0, unroll=4)

    def fire_tok(ch, s):
        pltpu.async_copy(tok_hbm.at[idx_all.at[pl.ds(ch * CHUNK, CHUNK)]],
                         rows[s], tsem[s])

    def wait_tok(s):
        pltpu.make_async_copy(tok_hbm.at[pl.ds(0, CHUNK)],
                              rows[s], tsem[s]).wait()

    def fire_spadd(ch, s):
        pltpu.async_copy(segpos_sh.at[spidx_all.at[pl.ds(ch * CHUNK, CHUNK)]],
                         rows[s], asem[s], add=True)

    def wait_spadd(s):
        pltpu.make_async_copy(segpos_hbm.at[pl.ds(0, CHUNK)],
                              rows[s], asem[s]).wait()

    def fire_out(ch, s):
        pltpu.async_copy(rows[s],
                         out_hbm.at[pl.ds(wbase + ch * CHUNK, CHUNK)],
                         osem[s])

    def wait_out(s):
        pltpu.make_async_copy(rows[s], out_hbm.at[pl.ds(0, CHUNK)],
                              osem[s]).wait()

    def steady(ch, s, sp, sf):
        wait_spadd(sp)
        fire_out(ch - 1, sp)
        wait_out(sf)
        fire_tok(ch + 4, sf)
        wait_tok(s)
        fire_spadd(ch, s)

    for c in range(4):
        fire_tok(c, c)
    wait_tok(0)
    fire_spadd(0, 0)
    fire_tok(4, 4)
    wait_spadd(0)
    fire_out(0, 0)
    wait_tok(1)
    fire_spadd(1, 1)
    fire_tok(5, 5)

    def six_body(c6, _):
        ch = 6 * c6 + 2
        for d in range(6):
            steady(ch + d, (2 + d) % NBUF, (1 + d) % NBUF, d % NBUF)
        return 0

    lax.fori_loop(0, 7, six_body, 0)

    steady(44, 44 % NBUF, 43 % NBUF, 48 % NBUF)
    steady(45, 45 % NBUF, 44 % NBUF, 49 % NBUF)

    for ch in (46, 47, 48, 49):
        wait_spadd((ch - 1) % NBUF)
        fire_out(ch - 1, (ch - 1) % NBUF)
        wait_tok(ch % NBUF)
        fire_spadd(ch, ch % NBUF)
    wait_spadd(49 % NBUF)
    fire_out(49, 49 % NBUF)
    for sl in (44, 45, 46, 47, 48, 49):
        wait_out(sl % NBUF)


@jax.jit
def kernel(input_ids, segment_ids, token_table, segment_table, pos_table):
    segpos = _build_segpos(segment_table, pos_table)
    mesh = plsc.VectorSubcoreMesh(core_axis_name="c", subcore_axis_name="s")
    kfn = pl.kernel(
        _sc_body,
        out_type=jax.ShapeDtypeStruct((N, H), jnp.float32),
        mesh=mesh,
        scratch_types=[
            pltpu.VMEM((PER_W,), jnp.int32),
            pltpu.VMEM((PER_W,), jnp.int32),
            pltpu.VMEM_SHARED((SP, H), jnp.float32),
        ] + [pltpu.VMEM((CHUNK, H), jnp.float32) for _ in range(NBUF)]
          + [pltpu.SemaphoreType.DMA for _ in range(3 * NBUF)],
    )
    out = kfn(input_ids.reshape(N).astype(jnp.int32),
              segment_ids.reshape(N).astype(jnp.int32),
              token_table, segpos)
    return out.reshape(B, S, H)


# speedup vs baseline: 1.0152x; 1.0152x over previous
import jax
import jax.numpy as jnp
from jax import lax
from jax.experimental import pallas as pl
from jax.experimental.pallas import tpu as pltpu
from jax.experimental.pallas import tpu_sc as plsc

B = 1024
S = 200
H = 128
N = B * S
LANES = 16
NUM_WORKERS = 32
PER_W = N // NUM_WORKERS
CHUNK = 128
NCHUNK = PER_W // CHUNK
NSEG = 3
SP = NSEG * S
NBUF = 6


def _segpos_tc_body(seg_ref, pos_ref, out_ref):
    for g in range(NSEG):
        out_ref[g * S:(g + 1) * S, :] = pos_ref[...] + seg_ref[g:g + 1, :]


def _build_segpos(segment_table, pos_table):
    return pl.pallas_call(
        _segpos_tc_body,
        out_shape=jax.ShapeDtypeStruct((SP, H), jnp.float32),
    )(segment_table, pos_table[:S])


def _sc_body(ids_hbm, sids_hbm, tok_hbm, segpos_hbm, out_hbm,
             idx_all, spidx_all, segpos_sh,
             rows_0, rows_1, rows_2, rows_3, rows_4, rows_5,
             tsem_0, tsem_1, tsem_2, tsem_3, tsem_4, tsem_5,
             asem_0, asem_1, asem_2, asem_3, asem_4, asem_5,
             osem_0, osem_1, osem_2, osem_3, osem_4, osem_5):
    info = plsc.get_sparse_core_info()
    nc = info.num_cores
    wid = lax.axis_index("s") * nc + lax.axis_index("c")
    wbase = wid * PER_W
    iota = lax.iota(jnp.int32, LANES)

    rows = (rows_0, rows_1, rows_2, rows_3, rows_4, rows_5)
    tsem = (tsem_0, tsem_1, tsem_2, tsem_3, tsem_4, tsem_5)
    asem = (asem_0, asem_1, asem_2, asem_3, asem_4, asem_5)
    osem = (osem_0, osem_1, osem_2, osem_3, osem_4, osem_5)

    pltpu.sync_copy(ids_hbm.at[pl.ds(wbase, PER_W)], idx_all)
    pltpu.sync_copy(sids_hbm.at[pl.ds(wbase, PER_W)], spidx_all)

    def fire_tok(ch, s):
        pltpu.async_copy(tok_hbm.at[idx_all.at[pl.ds(ch * CHUNK, CHUNK)]],
                         rows[s], tsem[s])

    def wait_tok(s):
        pltpu.make_async_copy(tok_hbm.at[pl.ds(0, CHUNK)],
                              rows[s], tsem[s]).wait()

    def fire_spadd(ch, s):
        pltpu.async_copy(segpos_sh.at[spidx_all.at[pl.ds(ch * CHUNK, CHUNK)]],
                         rows[s], asem[s], add=True)

    def wait_spadd(s):
        pltpu.make_async_copy(segpos_hbm.at[pl.ds(0, CHUNK)],
                              rows[s], asem[s]).wait()

    def fire_out(ch, s):
        pltpu.async_copy(rows[s],
                         out_hbm.at[pl.ds(wbase + ch * CHUNK, CHUNK)],
                         osem[s])

    def wait_out(s):
        pltpu.make_async_copy(rows[s], out_hbm.at[pl.ds(0, CHUNK)],
                              osem[s]).wait()

    def steady(ch, s, sp, sf):
        wait_spadd(sp)
        fire_out(ch - 1, sp)
        wait_out(sf)
        fire_tok(ch + 4, sf)
        wait_tok(s)
        fire_spadd(ch, s)

    for c in range(4):
        fire_tok(c, c)
    pltpu.sync_copy(segpos_hbm, segpos_sh)

    def spidx_body(i, _):
        sv = spidx_all[pl.ds(i * LANES, LANES)]
        pv = lax.rem(wbase + i * LANES + iota, S)
        spidx_all[pl.ds(i * LANES, LANES)] = sv * S + pv
        return 0

    lax.fori_loop(0, PER_W // LANES, spidx_body, 0, unroll=4)

    wait_tok(0)
    fire_spadd(0, 0)
    fire_tok(4, 4)
    wait_spadd(0)
    fire_out(0, 0)
    wait_tok(1)
    fire_spadd(1, 1)
    fire_tok(5, 5)

    def six_body(c6, _):
        ch = 6 * c6 + 2
        for d in range(6):
            steady(ch + d, (2 + d) % NBUF, (1 + d) % NBUF, d % NBUF)
        return 0

    lax.fori_loop(0, 7, six_body, 0)

    steady(44, 44 % NBUF, 43 % NBUF, 48 % NBUF)
    steady(45, 45 % NBUF, 44 % NBUF, 49 % NBUF)

    for ch in (46, 47, 48, 49):
        wait_spadd((ch - 1) % NBUF)
        fire_out(ch - 1, (ch - 1) % NBUF)
        wait_tok(ch % NBUF)
        fire_spadd(ch, ch % NBUF)
    wait_spadd(49 % NBUF)
    fire_out(49, 49 % NBUF)
    for sl in (44, 45, 46, 47, 48, 49):
        wait_out(sl % NBUF)


@jax.jit
def kernel(input_ids, segment_ids, token_table, segment_table, pos_table):
    segpos = _build_segpos(segment_table, pos_table)
    mesh = plsc.VectorSubcoreMesh(core_axis_name="c", subcore_axis_name="s")
    kfn = pl.kernel(
        _sc_body,
        out_type=jax.ShapeDtypeStruct((N, H), jnp.float32),
        mesh=mesh,
        scratch_types=[
            pltpu.VMEM((PER_W,), jnp.int32),
            pltpu.VMEM((PER_W,), jnp.int32),
            pltpu.VMEM_SHARED((SP, H), jnp.float32),
        ] + [pltpu.VMEM((CHUNK, H), jnp.float32) for _ in range(NBUF)]
          + [pltpu.SemaphoreType.DMA for _ in range(3 * NBUF)],
    )
    out = kfn(input_ids.reshape(N).astype(jnp.int32),
              segment_ids.reshape(N).astype(jnp.int32),
              token_table, segpos)
    return out.reshape(B, S, H)
